# Initial kernel scaffold; baseline (speedup 1.0000x reference)
import jax
import jax.numpy as jnp
from jax.experimental import pallas as pl


def kernel(hidden_activations):
    raise NotImplementedError("write your pallas kernel here")


# speedup vs baseline: 33.8119x; 33.8119x over previous
import functools

import jax
import jax.numpy as jnp
from jax import lax
from jax.experimental import pallas as pl
from jax.experimental.pallas import tpu as pltpu
from jax.experimental.pallas import tpu_sc as plsc

B, C, H, W = 8, 96, 224, 224
BC = B * C
PH, PW = H // 2, W // 2
NR = PH * PW
NREG = BC * NR
NRP = BC * PH

NC, NS = 2, 16
NW = NC * NS
RPW = NRP // NW
K = 32
NCHUNK = RPW // K

XCH = K * 448
GCH = K * 560
OCH = K * 112

_mesh = plsc.VectorSubcoreMesh(
    core_axis_name="c", subcore_axis_name="s", num_cores=NC, num_subcores=NS
)


@functools.partial(
    pl.kernel,
    mesh=_mesh,
    out_type=(
        jax.ShapeDtypeStruct((NRP * 448,), jnp.float32),
        jax.ShapeDtypeStruct((NREG,), jnp.float32),
        jax.ShapeDtypeStruct((NREG,), jnp.int32),
    ),
    scratch_types=[
        pltpu.VMEM((XCH,), jnp.float32),
        pltpu.VMEM((GCH,), jnp.float32),
        pltpu.VMEM((XCH,), jnp.float32),
        pltpu.VMEM((OCH,), jnp.float32),
        pltpu.VMEM((OCH,), jnp.int32),
    ],
    compiler_params=pltpu.CompilerParams(needs_layout_passes=False),
)
def _sc_pool(x_hbm, g_hbm, sp_hbm, po_hbm, wi_hbm, x_buf, g_buf, sp_buf, po_buf, wi_buf):
    wid = lax.axis_index("s") * NC + lax.axis_index("c")
    iota = lax.iota(jnp.int32, 16)
    iota2 = iota * 2
    iota5 = iota * 5

    def chunk_body(ci, _):
        row0 = wid * RPW + ci * K
        xbase = row0 * 448
        gbase = row0 * 560
        obase = row0 * 112
        pltpu.sync_copy(x_hbm.at[pl.ds(xbase, XCH)], x_buf)
        pltpu.sync_copy(g_hbm.at[pl.ds(gbase, GCH)], g_buf)

        def row_body(rp, _):
            xoff = rp * 448
            goff = rp * 560
            poff = rp * 112
            for tb in range(7):
                ia = iota2 + (xoff + 32 * tb)
                ib = ia + 1
                ic = ia + 224
                idd = ia + 225
                ig = iota5 + (goff + 80 * tb)
                a = plsc.load_gather(x_buf, [ia])
                b = plsc.load_gather(x_buf, [ib])
                c = plsc.load_gather(x_buf, [ic])
                d = plsc.load_gather(x_buf, [idd])
                ga = plsc.load_gather(g_buf, [ig])
                gb = plsc.load_gather(g_buf, [ig + 1])
                gc = plsc.load_gather(g_buf, [ig + 2])
                gd = plsc.load_gather(g_buf, [ig + 3])
                gn = plsc.load_gather(g_buf, [ig + 4])
                m = jnp.maximum(
                    jnp.maximum(jnp.maximum(a, b), jnp.maximum(c, d)), 0.0
                )
                ea = jnp.exp(a - m)
                eb = jnp.exp(b - m)
                ec = jnp.exp(c - m)
                ed = jnp.exp(d - m)
                en = jnp.exp(0.0 - m)
                s4 = ea + eb + ec + ed
                den = s4 + en + 1e-8
                rinv = 1.0 / den
                epsd = 1e-8 * den
                za = (ea + epsd) * ga
                zb = (eb + epsd) * gb
                zc = (ec + epsd) * gc
                zd = (ed + epsd) * gd
                zn = (en + epsd) * gn
                zm = jnp.maximum(
                    jnp.maximum(jnp.maximum(za, zb), jnp.maximum(zc, zd)), zn
                )
                ca = za == zm
                cb = zb == zm
                cc = zc == zm
                cd = zd == zm
                widx = jnp.where(
                    ca, 0, jnp.where(cb, 1, jnp.where(cc, 2, jnp.where(cd, 3, 4)))
                ).astype(jnp.int32)
                zero = jnp.zeros((16,), jnp.float32)
                ao = jnp.where(ca, ea * rinv, zero)
                bo = jnp.where(cb, eb * rinv, zero)
                co = jnp.where(cc, ec * rinv, zero)
                do = jnp.where(cd, ed * rinv, zero)
                pooled = jnp.minimum(jnp.maximum(s4 * rinv, 0.0), 1.0)
                plsc.store_scatter(sp_buf, [ia], ao)
                plsc.store_scatter(sp_buf, [ib], bo)
                plsc.store_scatter(sp_buf, [ic], co)
                plsc.store_scatter(sp_buf, [idd], do)
                po_buf[pl.ds(poff + 16 * tb, 16)] = pooled
                wi_buf[pl.ds(poff + 16 * tb, 16)] = widx
            return 0

        lax.fori_loop(0, K, row_body, 0)
        pltpu.sync_copy(sp_buf, sp_hbm.at[pl.ds(xbase, XCH)])
        pltpu.sync_copy(po_buf, po_hbm.at[pl.ds(obase, OCH)])
        pltpu.sync_copy(wi_buf, wi_hbm.at[pl.ds(obase, OCH)])
        return 0

    lax.fori_loop(0, NCHUNK, chunk_body, 0)


def kernel(hidden_activations):
    x_flat = hidden_activations.reshape(-1)
    u = jax.random.uniform(jax.random.key(42), (NREG * 5,), dtype=jnp.float32)
    g_flat = 1.0 / (-jnp.log(u + 1e-8) + 1e-8)
    sparse, pooled, winner = _sc_pool(x_flat, g_flat)
    sparse = sparse.reshape(B, C, H, W)
    pooled = pooled.reshape(B, C, PH, PW)
    winner = winner.reshape(B, C, PH, PW)
    return (sparse, pooled, winner)
